# baseline (speedup 1.0000x reference)
import dataclasses
import functools

import jax
import jax.numpy as jnp
from jax import lax
from jax.experimental import pallas as pl
from jax.experimental.pallas import tpu as pltpu
from jax.experimental.pallas import tpu_sc as plsc

_NUM_POINTS = 100000
_DIMS = 128
_BATCH = 16384
_SCALE_COEF = 1.0
_EPS = 1e-7

_NC = 2
_NS = 16
_L = 16
_NW = _NC * _NS
_BPW = _BATCH // _NW

_mesh = plsc.VectorSubcoreMesh(
    core_axis_name="c", subcore_axis_name="s", num_cores=_NC, num_subcores=_NS
)

_sc_params = pltpu.CompilerParams()
if "needs_layout_passes" in pltpu.CompilerParams.__dataclass_fields__:
    _sc_params = dataclasses.replace(_sc_params, needs_layout_passes=False)


_CHG = 128
_BUFN = 3


@functools.cache
def _make_sc_gather(n_pairs):
    bpw = n_pairs // _NW
    ntask = 2 * (bpw // _CHG)

    @functools.partial(
        pl.kernel,
        out_type=(
            jax.ShapeDtypeStruct((n_pairs, _DIMS), jnp.float32),
            jax.ShapeDtypeStruct((n_pairs, _DIMS), jnp.float32),
        ),
        mesh=_mesh,
        scratch_types=[
            pltpu.VMEM((bpw,), jnp.int32),
            pltpu.VMEM((bpw,), jnp.int32),
            pltpu.VMEM((_BUFN, _CHG, _DIMS), jnp.float32),
            [pltpu.SemaphoreType.DMA] * _BUFN,
            [pltpu.SemaphoreType.DMA] * _BUFN,
        ],
        compiler_params=_sc_params,
    )
    def sc_gather(
        emb_hbm, sidx_hbm, didx_hbm, src_out, dst_out,
        sidx_v, didx_v, bufs, sems_in, sems_out,
    ):
        wid = lax.axis_index("s") * _NC + lax.axis_index("c")
        base = wid * bpw
        pltpu.sync_copy(sidx_hbm.at[pl.ds(base, bpw)], sidx_v)
        pltpu.sync_copy(didx_hbm.at[pl.ds(base, bpw)], didx_v)

        def task(t):
            idx_v = sidx_v if t % 2 == 0 else didx_v
            out = src_out if t % 2 == 0 else dst_out
            return idx_v, out, (t // 2) * _CHG

        def issue_in(t):
            idx_v, _, off = task(t)
            b = t % _BUFN
            return pltpu.async_copy(
                emb_hbm.at[idx_v.at[pl.ds(off, _CHG)]], bufs.at[b], sems_in[b]
            )

        def issue_out(t):
            _, out, off = task(t)
            b = t % _BUFN
            return pltpu.async_copy(
                bufs.at[b], out.at[pl.ds(base + off, _CHG)], sems_out[b]
            )

        din = {}
        dout = {}
        ahead = min(_BUFN - 1, ntask)
        for t in range(ahead):
            din[t] = issue_in(t)
        unwaited = []
        for t in range(ntask):
            din[t].wait()
            dout[t] = issue_out(t)
            unwaited.append(t)
            nt = t + ahead
            if nt < ntask:
                prev = nt - _BUFN
                if prev >= 0:
                    dout[prev].wait()
                    unwaited.remove(prev)
                din[nt] = issue_in(nt)
        for t in unwaited:
            dout[t].wait()

    return sc_gather


_ROWS_PER_BLK = 8192
_NBLK = _BATCH // _ROWS_PER_BLK

_DN = (((1,), (1,)), ((), ()))


def _row_sums(prod, ones_bf):
    return jax.lax.dot_general(
        ones_bf, prod.astype(jnp.bfloat16), _DN,
        preferred_element_type=jnp.float32,
    )


def _tc_dist(src_ref, dst_ref, scale_ref, o_ref):
    s = src_ref[...]
    t = dst_ref[...]
    d = s - t
    ones_bf = jnp.ones((8, _DIMS), jnp.bfloat16)
    d2 = d * d
    d2h = d2.astype(jnp.bfloat16).astype(jnp.float32)
    ssq = _row_sums(s * s, ones_bf)[0:1]
    tsq = _row_sums(t * t, ones_bf)[0:1]
    sqd = (_row_sums(d2h, ones_bf) + _row_sums(d2 - d2h, ones_bf))[0:1]
    den = jnp.maximum(1.0 - ssq, _EPS) * jnp.maximum(1.0 - tsq, _EPS)
    arg = jnp.maximum(1.0 + 2.0 * sqd / den, 1.0 + _EPS)
    sval = jnp.maximum(scale_ref[0, 0] / _SCALE_COEF, 0.1)
    res = jnp.log(arg + jnp.sqrt((arg - 1.0) * (arg + 1.0))) * sval
    o_ref[...] = res.reshape(1, 1, _ROWS_PER_BLK)


_NCHUNK_PIPE = 1


def _tc_dist_call(src_e, dst_e, scale2):
    n = src_e.shape[0]
    nblk = n // _ROWS_PER_BLK
    dist = pl.pallas_call(
        _tc_dist,
        grid=(nblk,),
        in_specs=[
            pl.BlockSpec((_ROWS_PER_BLK, _DIMS), lambda i: (i, 0)),
            pl.BlockSpec((_ROWS_PER_BLK, _DIMS), lambda i: (i, 0)),
            pl.BlockSpec((1, 1), lambda i: (0, 0)),
        ],
        out_specs=pl.BlockSpec((1, 1, _ROWS_PER_BLK), lambda i: (i, 0, 0)),
        out_shape=jax.ShapeDtypeStruct((nblk, 1, _ROWS_PER_BLK), jnp.float32),
    )(src_e, dst_e, scale2)
    return dist.reshape(n)


def kernel(input_triplet, embeddings, scale):
    trip = input_triplet.astype(jnp.int32)
    sidx = trip[:, 0]
    didx = trip[:, 1]
    scale2 = scale.reshape(1, 1)
    csz = _BATCH // _NCHUNK_PIPE
    sc_gather = _make_sc_gather(csz)
    outs = []
    for c in range(_NCHUNK_PIPE):
        src_e, dst_e = sc_gather(embeddings, sidx[c * csz:(c + 1) * csz],
                                 didx[c * csz:(c + 1) * csz])
        outs.append(_tc_dist_call(src_e, dst_e, scale2))
    if _NCHUNK_PIPE == 1:
        return outs[0]
    return jnp.concatenate(outs)


# speedup vs baseline: 1.0057x; 1.0057x over previous
import dataclasses
import functools

import jax
import jax.numpy as jnp
from jax import lax
from jax.experimental import pallas as pl
from jax.experimental.pallas import tpu as pltpu
from jax.experimental.pallas import tpu_sc as plsc

_NUM_POINTS = 100000
_DIMS = 128
_BATCH = 16384
_SCALE_COEF = 1.0
_EPS = 1e-7

_NC = 2
_NS = 16
_L = 16
_NW = _NC * _NS
_BPW = _BATCH // _NW

_mesh = plsc.VectorSubcoreMesh(
    core_axis_name="c", subcore_axis_name="s", num_cores=_NC, num_subcores=_NS
)

_sc_params = pltpu.CompilerParams()
if "needs_layout_passes" in pltpu.CompilerParams.__dataclass_fields__:
    _sc_params = dataclasses.replace(_sc_params, needs_layout_passes=False)


_CHG = 128
_BUFN = 3


@functools.cache
def _make_sc_gather(n_pairs):
    bpw = n_pairs // _NW
    ntask = 2 * (bpw // _CHG)

    @functools.partial(
        pl.kernel,
        out_type=(
            jax.ShapeDtypeStruct((n_pairs, _DIMS), jnp.float32),
            jax.ShapeDtypeStruct((n_pairs, _DIMS), jnp.float32),
        ),
        mesh=_mesh,
        scratch_types=[
            pltpu.VMEM((bpw,), jnp.int32),
            pltpu.VMEM((bpw,), jnp.int32),
            pltpu.VMEM((_BUFN, _CHG, _DIMS), jnp.float32),
            [pltpu.SemaphoreType.DMA] * _BUFN,
            [pltpu.SemaphoreType.DMA] * _BUFN,
        ],
        compiler_params=_sc_params,
    )
    def sc_gather(
        emb_hbm, sidx_hbm, didx_hbm, src_out, dst_out,
        sidx_v, didx_v, bufs, sems_in, sems_out,
    ):
        wid = lax.axis_index("s") * _NC + lax.axis_index("c")
        base = wid * bpw
        pltpu.sync_copy(sidx_hbm.at[pl.ds(base, bpw)], sidx_v)
        pltpu.sync_copy(didx_hbm.at[pl.ds(base, bpw)], didx_v)

        def task(t):
            idx_v = sidx_v if t % 2 == 0 else didx_v
            out = src_out if t % 2 == 0 else dst_out
            return idx_v, out, (t // 2) * _CHG

        def issue_in(t):
            idx_v, _, off = task(t)
            b = t % _BUFN
            return pltpu.async_copy(
                emb_hbm.at[idx_v.at[pl.ds(off, _CHG)]], bufs.at[b], sems_in[b]
            )

        def issue_out(t):
            _, out, off = task(t)
            b = t % _BUFN
            return pltpu.async_copy(
                bufs.at[b], out.at[pl.ds(base + off, _CHG)], sems_out[b]
            )

        din = {}
        dout = {}
        ahead = min(_BUFN - 1, ntask)
        for t in range(ahead):
            din[t] = issue_in(t)
        unwaited = []
        for t in range(ntask):
            din[t].wait()
            dout[t] = issue_out(t)
            unwaited.append(t)
            nt = t + ahead
            if nt < ntask:
                prev = nt - _BUFN
                if prev >= 0:
                    dout[prev].wait()
                    unwaited.remove(prev)
                din[nt] = issue_in(nt)
        for t in unwaited:
            dout[t].wait()

    return sc_gather


_ROWS_PER_BLK = 4096
_NBLK = _BATCH // _ROWS_PER_BLK

_DN = (((1,), (1,)), ((), ()))


def _row_sums(prod, ones_bf):
    return jax.lax.dot_general(
        ones_bf, prod.astype(jnp.bfloat16), _DN,
        preferred_element_type=jnp.float32,
    )


def _tc_dist(src_ref, dst_ref, scale_ref, o_ref):
    s = src_ref[...]
    t = dst_ref[...]
    d = s - t
    ones_bf = jnp.ones((8, _DIMS), jnp.bfloat16)
    d2 = d * d
    d2h = d2.astype(jnp.bfloat16).astype(jnp.float32)
    ssq = _row_sums(s * s, ones_bf)[0:1]
    tsq = _row_sums(t * t, ones_bf)[0:1]
    sqd = (_row_sums(d2h, ones_bf) + _row_sums(d2 - d2h, ones_bf))[0:1]
    den = jnp.maximum(1.0 - ssq, _EPS) * jnp.maximum(1.0 - tsq, _EPS)
    arg = jnp.maximum(1.0 + 2.0 * sqd / den, 1.0 + _EPS)
    sval = jnp.maximum(scale_ref[0, 0] / _SCALE_COEF, 0.1)
    res = jnp.log(arg + jnp.sqrt((arg - 1.0) * (arg + 1.0))) * sval
    o_ref[...] = res.reshape(1, 1, _ROWS_PER_BLK)


def _tc_dist_call(src_e, dst_e, scale2):
    n = src_e.shape[0]
    nblk = n // _ROWS_PER_BLK
    dist = pl.pallas_call(
        _tc_dist,
        grid=(nblk,),
        in_specs=[
            pl.BlockSpec((_ROWS_PER_BLK, _DIMS), lambda i: (i, 0)),
            pl.BlockSpec((_ROWS_PER_BLK, _DIMS), lambda i: (i, 0)),
            pl.BlockSpec((1, 1), lambda i: (0, 0)),
        ],
        out_specs=pl.BlockSpec((1, 1, _ROWS_PER_BLK), lambda i: (i, 0, 0)),
        out_shape=jax.ShapeDtypeStruct((nblk, 1, _ROWS_PER_BLK), jnp.float32),
    )(src_e, dst_e, scale2)
    return dist.reshape(n)


def kernel(input_triplet, embeddings, scale):
    trip = input_triplet.astype(jnp.int32)
    src_e, dst_e = _make_sc_gather(_BATCH)(embeddings, trip[:, 0], trip[:, 1])
    return _tc_dist_call(src_e, dst_e, scale.reshape(1, 1))
